# baseline (speedup 1.0000x reference)
import jax
import jax.numpy as jnp
from jax.experimental import pallas as pl
from jax.experimental.pallas import tpu as pltpu

_IB = 1


def _fused_body(x_ref, o1_ref, o2_ref, o3_ref, o4_ref,
                a1_s, a2_s, b1_s, b2_s,
                w1_s, w2_s, w3_s, w4_s):
    @pl.when(pl.program_id(0) == 0)
    def _prep():
        x = x_ref[...]

        def cls(base):
            return x[:, base : base + 1, :] * x[:, base + 1 : base + 21, :]

        a1_s[...] = jnp.transpose(cls(0), (2, 0, 1))
        a2_s[...] = jnp.transpose(cls(51), (2, 0, 1))
        b1_s[...] = jnp.transpose(0.5 * cls(102), (2, 0, 1))
        b2_s[...] = jnp.transpose(0.5 * cls(153), (2, 0, 1))

        s_row = jax.lax.broadcasted_iota(jnp.int32, (14, 196), 0)
        p_col = jax.lax.broadcasted_iota(jnp.int32, (14, 196), 1)
        Rt = (p_col // 14 == s_row).astype(jnp.float32)
        Tt = (p_col % 14 == s_row).astype(jnp.float32)

        def sel(slab, onehot):
            return jax.lax.dot_general(
                slab, onehot, (((1,), (0,)), ((), ())),
                preferred_element_type=jnp.float32,
                precision=jax.lax.Precision.HIGHEST,
            )

        def lc_all(xbase):
            cx = jnp.transpose(x[:, xbase + 23 : xbase + 37, :], (2, 0, 1))
            cy = jnp.transpose(x[:, xbase + 37 : xbase + 51, :], (2, 0, 1))
            lx = sel(cx.reshape(196 * 16, 14), Rt)
            ly = sel(cy.reshape(196 * 16, 14), Tt)
            return (lx * ly).reshape(196, 16, 196)

        def lz_all(xbase):
            zx = jnp.transpose(x[:, xbase + 23 : xbase + 37, :], (1, 0, 2))
            zy = jnp.transpose(x[:, xbase + 37 : xbase + 51, :], (1, 0, 2))
            zxr = jnp.broadcast_to(zx[:, None], (14, 14, 16, 196)).reshape(196, 16, 196)
            zyr = jnp.broadcast_to(zy[None], (14, 14, 16, 196)).reshape(196, 16, 196)
            return zxr * zyr

        Lz1 = lz_all(102)
        w1_s[...] = Lz1
        w2_s[...] = Lz1
        Lz2 = lz_all(153)
        w3_s[...] = Lz2
        w4_s[...] = Lz2
        Lc1 = lc_all(0)
        w1_s[...] = Lc1 * w1_s[...]
        w3_s[...] = Lc1 * w3_s[...]
        Lc2 = lc_all(51)
        w2_s[...] = Lc2 * w2_s[...]
        w4_s[...] = Lc2 * w4_s[...]

    B1 = b1_s[...]
    B2 = b2_s[...]
    i0 = pl.program_id(0) * _IB
    for k in range(_IB):
        i = i0 + k
        A1 = a1_s[i]
        A2 = a2_s[i]

        def emit(o_ref, w_s, A, B):
            WT = jnp.transpose(w_s[i])
            o_ref[k] = (WT[:, :, None] * A[None, :, :]) * B

        emit(o1_ref, w1_s, A1, B1)
        emit(o2_ref, w2_s, A2, B1)
        emit(o3_ref, w3_s, A1, B2)
        emit(o4_ref, w4_s, A2, B2)


def kernel(inference):
    N = inference.shape[0]
    inf = inference.reshape(N, 204, 196)
    f32 = jnp.float32
    scratch = (
        [pltpu.VMEM((196, 16, 20), f32)] * 4
        + [pltpu.VMEM((196, 16, 196), f32)] * 4
    )
    outs = pl.pallas_call(
        _fused_body,
        grid=(196 // _IB,),
        in_specs=[pl.BlockSpec((N, 204, 196), lambda i: (0, 0, 0))],
        out_specs=[pl.BlockSpec((_IB, 196, 16, 20), lambda i: (i, 0, 0, 0))] * 4,
        out_shape=[jax.ShapeDtypeStruct((196, 196, 16, 20), f32)] * 4,
        scratch_shapes=scratch,
        compiler_params=pltpu.CompilerParams(
            dimension_semantics=("arbitrary",),
        ),
    )(inf)
    return tuple(
        jnp.transpose(o, (2, 3, 0, 1)).reshape(N, 20, 14, 14, 14, 14) for o in outs
    )
